# baseline (speedup 1.0000x reference)
import jax
import jax.numpy as jnp
from jax.experimental import pallas as pl
from jax.experimental.pallas import tpu as pltpu

_VERTEX_IDS = (9120, 9929, 9448, 616, 6,
               5770, 5780, 8846, 8463, 8474, 8635,
               5361, 4933, 5058, 5169, 5286,
               8079, 7669, 7794, 7905, 8022)

_B = 1024
_V = 10475
_J = 55
_E = len(_VERTEX_IDS)


def _body(jt_hbm, vt_hbm, out_ref, jbuf, sbuf, jsem, ssem):
    jcopy = pltpu.make_async_copy(jt_hbm, jbuf, jsem)
    jcopy.start(priority=1)
    scopies = [
        pltpu.make_async_copy(
            vt_hbm.at[:, pl.ds(8 * (idx // 8), 8), :], sbuf.at[j], ssem.at[j])
        for j, idx in enumerate(_VERTEX_IDS)
    ]
    for j, cp in enumerate(scopies):
        cp.start(priority=1 if j % 3 == 2 else 0)
    jcopy.wait()
    out_ref[:, : _J, :] = jbuf[...]
    for j, idx in enumerate(_VERTEX_IDS):
        scopies[j].wait()
        out_ref[:, _J + j, :] = sbuf[j, :, idx % 8, :]


def kernel(vertices, joints):
    vt = vertices.transpose(2, 1, 0)
    jt = joints.transpose(2, 1, 0)

    out_t = pl.pallas_call(
        _body,
        grid=(1,),
        out_shape=jax.ShapeDtypeStruct((3, _J + _E, _B), jnp.float32),
        in_specs=[
            pl.BlockSpec(memory_space=pl.ANY),
            pl.BlockSpec(memory_space=pl.ANY),
        ],
        out_specs=pl.BlockSpec((3, _J + _E, _B), lambda i: (0, 0, 0)),
        scratch_shapes=[
            pltpu.VMEM((3, _J, _B), jnp.float32),
            pltpu.VMEM((_E, 3, 8, _B), jnp.float32),
            pltpu.SemaphoreType.DMA,
            pltpu.SemaphoreType.DMA((_E,)),
        ],
    )(jt, vt)
    return out_t.transpose(2, 1, 0)
